# baseline (speedup 1.0000x reference)
import functools

import jax
import jax.numpy as jnp
from jax import lax
from jax.experimental import pallas as pl
from jax.experimental.pallas import tpu as pltpu
from jax.experimental.pallas import tpu_sc as plsc

VOCAB = 100000
N_EMBD = 768
BATCH = 4
SEQ_LEN = 2048

_LANES = 16
_NC = 2
_NS = 16
_NW = _NC * _NS

_S_PER_W = SEQ_LEN // _NW
_G = 16
_NG = _S_PER_W // _G
_ROW_SLICES = N_EMBD // _LANES


def _emb_body(ids_hbm, table_hbm, pos_hbm, out_hbm,
              idx_v, tok_v, pos_v, gsem, osem, psem, isem):
    wid = lax.axis_index("s") * _NC + lax.axis_index("c")
    s_base = wid * _S_PER_W

    id_cps = []
    for b in range(BATCH):
        id_cps.append(pltpu.async_copy(
            ids_hbm.at[b, pl.ds(pl.multiple_of(s_base, _S_PER_W), _S_PER_W)],
            idx_v.at[b], isem))
    pos_cp = [pltpu.async_copy(
        pos_hbm.at[pl.ds(pl.multiple_of(s_base, _S_PER_W), _G)],
        pos_v.at[0], psem.at[0])]
    for cp in id_cps:
        cp.wait()

    def issue_gathers(g, slot):
        cps = []
        for b in range(BATCH):
            idx = idx_v.at[b, pl.ds(g * _G, _G)]
            cps.append(pltpu.async_copy(table_hbm.at[idx], tok_v.at[slot, b],
                                        gsem.at[slot]))
        return cps

    gather_cp = [None, None]
    out_cp = [None, None]
    gather_cp[0] = issue_gathers(0, 0)
    for g in range(_NG):
        slot = g % 2
        nxt = 1 - slot
        if g + 1 < _NG:
            if out_cp[nxt] is not None:
                for cp in out_cp[nxt]:
                    cp.wait()
                out_cp[nxt] = None
            s_off = pl.multiple_of(s_base + (g + 1) * _G, _G)
            pos_cp.append(pltpu.async_copy(pos_hbm.at[pl.ds(s_off, _G)],
                                           pos_v.at[nxt], psem.at[nxt]))
            gather_cp[nxt] = issue_gathers(g + 1, nxt)
        pos_cp.pop(0).wait()
        for cp in gather_cp[slot]:
            cp.wait()

        s_off = pl.multiple_of(s_base + g * _G, _G)
        half = _G // 2
        cps = []
        for hh in range(2):
            @plsc.parallel_loop(hh * half, (hh + 1) * half)
            def add_row(r):
                for j in range(_ROW_SLICES):
                    sl = pl.ds(j * _LANES, _LANES)
                    p = pos_v[slot, r, sl]
                    for b in range(BATCH):
                        plsc.addupdate(tok_v.at[slot, b, r, sl], p)

            cps.append(pltpu.async_copy(
                tok_v.at[slot, :, pl.ds(hh * half, half), :],
                out_hbm.at[:, pl.ds(s_off + hh * half, half), :],
                osem.at[slot]))
        out_cp[slot] = cps
    for cps_ in out_cp:
        if cps_ is not None:
            for cp in cps_:
                cp.wait()


@jax.jit
def _emb_call(input_ids, token_table, position_table):
    mesh = plsc.VectorSubcoreMesh(core_axis_name="c", subcore_axis_name="s")
    k = functools.partial(
        pl.kernel,
        out_type=jax.ShapeDtypeStruct((BATCH, SEQ_LEN, N_EMBD), jnp.float32),
        mesh=mesh,
        scratch_types=[
            pltpu.VMEM((BATCH, _S_PER_W), jnp.int32),
            pltpu.VMEM((2, BATCH, _G, N_EMBD), jnp.float32),
            pltpu.VMEM((2, _G, N_EMBD), jnp.float32),
            pltpu.SemaphoreType.DMA((2,)),
            pltpu.SemaphoreType.DMA((2,)),
            pltpu.SemaphoreType.DMA((2,)),
            pltpu.SemaphoreType.DMA,
        ],
    )(_emb_body)
    return k(input_ids, token_table, position_table)


def kernel(input_ids, token_table, position_table):
    return _emb_call(input_ids.astype(jnp.int32), token_table, position_table)


# speedup vs baseline: 1.0470x; 1.0470x over previous
import functools

import jax
import jax.numpy as jnp
from jax import lax
from jax.experimental import pallas as pl
from jax.experimental.pallas import tpu as pltpu
from jax.experimental.pallas import tpu_sc as plsc

VOCAB = 100000
N_EMBD = 768
BATCH = 4
SEQ_LEN = 2048

_LANES = 16
_NC = 2
_NS = 16
_NW = _NC * _NS

_S_PER_W = SEQ_LEN // _NW
_G = 16
_NG = _S_PER_W // _G
_ROW_SLICES = N_EMBD // _LANES


def _emb_body(ids_hbm, table_hbm, pos_hbm, out_hbm,
              idx_v, tok_v, pos_v, gsem, osem, psem, isem):
    wid = lax.axis_index("s") * _NC + lax.axis_index("c")
    s_base = wid * _S_PER_W

    id_cps = []
    for b in range(BATCH):
        id_cps.append(pltpu.async_copy(
            ids_hbm.at[b, pl.ds(pl.multiple_of(s_base, _S_PER_W), _S_PER_W)],
            idx_v.at[b], isem))
    pos_cp = [pltpu.async_copy(
        pos_hbm.at[pl.ds(pl.multiple_of(s_base, _S_PER_W), _G)],
        pos_v.at[0], psem.at[0])]
    for cp in id_cps:
        cp.wait()

    def issue_gathers(g, slot):
        cps = []
        for b in range(BATCH):
            idx = idx_v.at[b, pl.ds(g * _G, _G)]
            cps.append(pltpu.async_copy(table_hbm.at[idx], tok_v.at[slot, b],
                                        gsem.at[slot]))
        return cps

    gather_cp = [None, None]
    out_cp = [None, None]
    gather_cp[0] = issue_gathers(0, 0)
    for g in range(_NG):
        slot = g % 2
        nxt = 1 - slot
        if g + 1 < _NG:
            if out_cp[nxt] is not None:
                for cp in out_cp[nxt]:
                    cp.wait()
                out_cp[nxt] = None
            s_off = pl.multiple_of(s_base + (g + 1) * _G, _G)
            pos_cp.append(pltpu.async_copy(pos_hbm.at[pl.ds(s_off, _G)],
                                           pos_v.at[nxt], psem.at[nxt]))
            gather_cp[nxt] = issue_gathers(g + 1, nxt)
        pos_cp.pop(0).wait()
        for cp in gather_cp[slot]:
            cp.wait()

        @plsc.parallel_loop(0, _G)
        def add_row(r):
            for j in range(_ROW_SLICES):
                sl = pl.ds(j * _LANES, _LANES)
                p = pos_v[slot, r, sl]
                for b in range(BATCH):
                    plsc.addupdate(tok_v.at[slot, b, r, sl], p)

        s_off = pl.multiple_of(s_base + g * _G, _G)
        out_cp[slot] = [pltpu.async_copy(
            tok_v.at[slot], out_hbm.at[:, pl.ds(s_off, _G), :],
            osem.at[slot])]
    for cps_ in out_cp:
        if cps_ is not None:
            for cp in cps_:
                cp.wait()


@jax.jit
def _emb_call(input_ids, token_table, position_table):
    mesh = plsc.VectorSubcoreMesh(core_axis_name="c", subcore_axis_name="s")
    k = functools.partial(
        pl.kernel,
        out_type=jax.ShapeDtypeStruct((BATCH, SEQ_LEN, N_EMBD), jnp.float32),
        mesh=mesh,
        scratch_types=[
            pltpu.VMEM((BATCH, _S_PER_W), jnp.int32),
            pltpu.VMEM((2, BATCH, _G, N_EMBD), jnp.float32),
            pltpu.VMEM((2, _G, N_EMBD), jnp.float32),
            pltpu.SemaphoreType.DMA((2,)),
            pltpu.SemaphoreType.DMA((2,)),
            pltpu.SemaphoreType.DMA((2,)),
            pltpu.SemaphoreType.DMA,
        ],
    )(_emb_body)
    return k(input_ids, token_table, position_table)


def kernel(input_ids, token_table, position_table):
    return _emb_call(input_ids.astype(jnp.int32), token_table, position_table)
